# Initial kernel scaffold; baseline (speedup 1.0000x reference)
import jax
import jax.numpy as jnp
from jax import lax
from jax.experimental import pallas as pl
from jax.experimental.pallas import tpu as pltpu
from jax.experimental.pallas import tpu_sc as plsc

NC, NS, LANES = 2, 16, 16
NW = NC * NS
CH = 512


def _emb_body(x_hbm, tab_hbm, pos_hbm, out_hbm, idx_v, rows_v, pos_v, gsem):
    wid = lax.axis_index("s") * NC + lax.axis_index("c")
    n_rows = out_hbm.shape[0]
    rows_per_w = n_rows // NW
    nchunks = rows_per_w // CH
    base = wid * rows_per_w

    pltpu.sync_copy(pos_hbm, pos_v)

    def chunk_body(c, carry):
        r0 = base + c * CH
        pltpu.sync_copy(x_hbm.at[pl.ds(r0, CH)], idx_v)
        pltpu.async_copy(tab_hbm.at[idx_v], rows_v, gsem).wait()

        def add_body(r, carry2):
            for k in range(4):
                sl = pl.ds(k * LANES, LANES)
                rows_v[r, sl] = rows_v[r, sl] + pos_v[r, sl]
            return carry2

        lax.fori_loop(0, CH, add_body, 0)
        pltpu.sync_copy(rows_v, out_hbm.at[pl.ds(r0, CH)])
        return carry

    lax.fori_loop(0, nchunks, chunk_body, 0)


def kernel(x, embed_w, pos_w):
    batch, seq = x.shape
    _, emb = embed_w.shape
    x_flat = x.reshape(-1).astype(jnp.int32)
    mesh = plsc.VectorSubcoreMesh(
        core_axis_name="c", subcore_axis_name="s",
        num_cores=NC, num_subcores=NS,
    )
    out = pl.kernel(
        _emb_body,
        out_type=jax.ShapeDtypeStruct((batch * seq, emb), jnp.float32),
        mesh=mesh,
        scratch_types=[
            pltpu.VMEM((CH,), jnp.int32),
            pltpu.VMEM((CH, emb), jnp.float32),
            pltpu.VMEM((seq, emb), jnp.float32),
            pltpu.SemaphoreType.DMA,
        ],
    )(x_flat, embed_w, pos_w)
    return out.reshape(batch, seq, emb)


# speedup vs baseline: 3.3607x; 3.3607x over previous
import jax
import jax.numpy as jnp
from jax import lax
from jax.experimental import pallas as pl
from jax.experimental.pallas import tpu as pltpu
from jax.experimental.pallas import tpu_sc as plsc

NC, NS, LANES = 2, 16, 16
NW = NC * NS
CH = 512


def _emb_body(x_hbm, tab_hbm, pos_hbm, out_hbm, idx_v, rows_v, pos_v, gsem):
    wid = lax.axis_index("s") * NC + lax.axis_index("c")
    n_rows = out_hbm.shape[0]
    rows_per_w = n_rows // NW
    nchunks = rows_per_w // CH
    base = wid * rows_per_w

    pltpu.sync_copy(pos_hbm, pos_v)

    def chunk_body(c, carry):
        r0 = base + c * CH
        pltpu.sync_copy(x_hbm.at[pl.ds(r0, CH)], idx_v)
        pltpu.async_copy(tab_hbm.at[idx_v], rows_v, gsem).wait()

        def add_body(r, carry2):
            for k in range(4):
                sl = pl.ds(k * LANES, LANES)
                rows_v[r, sl] = rows_v[r, sl] + pos_v[r, sl]
            return carry2

        lax.fori_loop(0, CH, add_body, 0)
        pltpu.sync_copy(rows_v, out_hbm.at[pl.ds(r0, CH)])
        return carry

    lax.fori_loop(0, nchunks, chunk_body, 0)


def kernel(x, embed_w, pos_w):
    batch, seq = x.shape
    _, emb = embed_w.shape
    x_flat = x.reshape(-1).astype(jnp.int32)
    mesh = plsc.VectorSubcoreMesh(
        core_axis_name="c", subcore_axis_name="s",
        num_cores=NC, num_subcores=NS,
    )
    out = pl.kernel(
        _emb_body,
        out_type=jax.ShapeDtypeStruct((batch * seq, emb), jnp.float32),
        mesh=mesh,
        compiler_params=pltpu.CompilerParams(use_tc_tiling_on_sc=False),
        scratch_types=[
            pltpu.VMEM((CH,), jnp.int32),
            pltpu.VMEM((CH, emb), jnp.float32),
            pltpu.VMEM((seq, emb), jnp.float32),
            pltpu.SemaphoreType.DMA,
        ],
    )(x_flat, embed_w, pos_w)
    return out.reshape(batch, seq, emb)


# speedup vs baseline: 3.9182x; 1.1659x over previous
import jax
import jax.numpy as jnp
from jax import lax
from jax.experimental import pallas as pl
from jax.experimental.pallas import tpu as pltpu
from jax.experimental.pallas import tpu_sc as plsc

NC, NS, LANES = 2, 16, 16
NW = NC * NS
CH = 256
NBUF = 4
LOOKAHEAD = 2


def _emb_body(x_hbm, tab_hbm, pos_hbm, out_hbm,
              i0, i1, i2, i3, r0, r1, r2, r3, pos_v,
              g0, g1, g2, g3, w0, w1, w2, w3):
    idx = [i0, i1, i2, i3]
    rows = [r0, r1, r2, r3]
    gsem = [g0, g1, g2, g3]
    wsem = [w0, w1, w2, w3]

    wid = lax.axis_index("s") * NC + lax.axis_index("c")
    n_rows = out_hbm.shape[0]
    rows_per_w = n_rows // NW
    nchunks = rows_per_w // CH
    base = wid * rows_per_w
    seq = pos_hbm.shape[0]
    per_seq = seq // CH

    pltpu.sync_copy(pos_hbm, pos_v)

    for b in range(LOOKAHEAD):
        pltpu.sync_copy(x_hbm.at[pl.ds(base + b * CH, CH)], idx[b])
        pltpu.async_copy(tab_hbm.at[idx[b]], rows[b], gsem[b])

    def ring(i, carry):
        for b in range(NBUF):
            c = i * NBUF + b
            s = b
            sp = (b + LOOKAHEAD) % NBUF

            @pl.when(c + LOOKAHEAD < nchunks)
            def _():
                @pl.when(c >= NBUF - LOOKAHEAD)
                def _():
                    cc = c + LOOKAHEAD - NBUF
                    pltpu.make_async_copy(
                        rows[sp], out_hbm.at[pl.ds(base + cc * CH, CH)],
                        wsem[sp]).wait()
                cg = c + LOOKAHEAD
                pltpu.sync_copy(x_hbm.at[pl.ds(base + cg * CH, CH)], idx[sp])
                pltpu.async_copy(tab_hbm.at[idx[sp]], rows[sp], gsem[sp])

            pltpu.make_async_copy(tab_hbm.at[idx[s]], rows[s], gsem[s]).wait()
            poff = (b % per_seq) * CH

            def add_body(r, carry2):
                for k in range(4):
                    sl = pl.ds(k * LANES, LANES)
                    plsc.addupdate(rows[s].at[r, sl], pos_v[poff + r, sl])
                return carry2

            lax.fori_loop(0, CH, add_body, 0)
            pltpu.async_copy(rows[s], out_hbm.at[pl.ds(base + c * CH, CH)],
                             wsem[s])
        return carry

    lax.fori_loop(0, nchunks // NBUF, ring, 0)

    for b in range(NBUF):
        cc = nchunks - NBUF + b
        pltpu.make_async_copy(
            rows[b], out_hbm.at[pl.ds(base + cc * CH, CH)], wsem[b]).wait()


def kernel(x, embed_w, pos_w):
    batch, seq = x.shape
    _, emb = embed_w.shape
    x_flat = x.reshape(-1).astype(jnp.int32)
    mesh = plsc.VectorSubcoreMesh(
        core_axis_name="c", subcore_axis_name="s",
        num_cores=NC, num_subcores=NS,
    )
    out = pl.kernel(
        _emb_body,
        out_type=jax.ShapeDtypeStruct((batch * seq, emb), jnp.float32),
        mesh=mesh,
        compiler_params=pltpu.CompilerParams(use_tc_tiling_on_sc=False),
        scratch_types=(
            [pltpu.VMEM((CH,), jnp.int32) for _ in range(NBUF)]
            + [pltpu.VMEM((CH, emb), jnp.float32) for _ in range(NBUF)]
            + [pltpu.VMEM((seq, emb), jnp.float32)]
            + [pltpu.SemaphoreType.DMA for _ in range(2 * NBUF)]
        ),
    )(x_flat, embed_w, pos_w)
    return out.reshape(batch, seq, emb)


# speedup vs baseline: 3.9228x; 1.0012x over previous
import jax
import jax.numpy as jnp
from jax import lax
from jax.experimental import pallas as pl
from jax.experimental.pallas import tpu as pltpu
from jax.experimental.pallas import tpu_sc as plsc

NC, NS, LANES = 2, 16, 16
NW = NC * NS
CH = 256
NBUF = 4
LOOKAHEAD = 2


def _emb_body(x_hbm, tab_hbm, pos_hbm, out_hbm,
              i0, i1, i2, i3, r0, r1, r2, r3, pos_v,
              g0, g1, g2, g3, w0, w1, w2, w3):
    idx = [i0, i1, i2, i3]
    rows = [r0, r1, r2, r3]
    gsem = [g0, g1, g2, g3]
    wsem = [w0, w1, w2, w3]

    wid = lax.axis_index("s") * NC + lax.axis_index("c")
    batch = x_hbm.shape[0]
    seq = pos_hbm.shape[0]
    per_seq = seq // CH
    batches_per_w = batch // NW
    nchunks = batches_per_w * per_seq
    base_b = wid * batches_per_w

    def chunk_slices(c, b):
        bb = base_b + c // per_seq
        h = b % per_seq
        return bb, h * CH

    pltpu.sync_copy(pos_hbm, pos_v)

    for b in range(LOOKAHEAD):
        bb, off = chunk_slices(b, b)
        pltpu.sync_copy(x_hbm.at[bb, pl.ds(off, CH)], idx[b])
        pltpu.async_copy(tab_hbm.at[idx[b]], rows[b], gsem[b])

    def ring(i, carry):
        for b in range(NBUF):
            c = i * NBUF + b
            s = b
            sp = (b + LOOKAHEAD) % NBUF

            @pl.when(c + LOOKAHEAD < nchunks)
            def _():
                @pl.when(c >= NBUF - LOOKAHEAD)
                def _():
                    cc = c + LOOKAHEAD - NBUF
                    wb, woff = chunk_slices(cc, (b + LOOKAHEAD) % NBUF)
                    pltpu.make_async_copy(
                        rows[sp], out_hbm.at[wb, pl.ds(woff, CH)],
                        wsem[sp]).wait()
                gb, goff = chunk_slices(c + LOOKAHEAD, (b + LOOKAHEAD) % NBUF)
                pltpu.sync_copy(x_hbm.at[gb, pl.ds(goff, CH)], idx[sp])
                pltpu.async_copy(tab_hbm.at[idx[sp]], rows[sp], gsem[sp])

            pltpu.make_async_copy(tab_hbm.at[idx[s]], rows[s], gsem[s]).wait()
            poff = (b % per_seq) * CH

            def add_body(r, carry2):
                for k in range(4):
                    sl = pl.ds(k * LANES, LANES)
                    plsc.addupdate(rows[s].at[r, sl], pos_v[poff + r, sl])
                return carry2

            lax.fori_loop(0, CH, add_body, 0)
            ob, ooff = chunk_slices(c, b)
            pltpu.async_copy(rows[s], out_hbm.at[ob, pl.ds(ooff, CH)],
                             wsem[s])
        return carry

    lax.fori_loop(0, nchunks // NBUF, ring, 0)

    for b in range(NBUF):
        cc = nchunks - NBUF + b
        ob, ooff = chunk_slices(cc, b)
        pltpu.make_async_copy(
            rows[b], out_hbm.at[ob, pl.ds(ooff, CH)], wsem[b]).wait()


def kernel(x, embed_w, pos_w):
    batch, seq = x.shape
    _, emb = embed_w.shape
    mesh = plsc.VectorSubcoreMesh(
        core_axis_name="c", subcore_axis_name="s",
        num_cores=NC, num_subcores=NS,
    )
    out = pl.kernel(
        _emb_body,
        out_type=jax.ShapeDtypeStruct((batch, seq, emb), jnp.float32),
        mesh=mesh,
        compiler_params=pltpu.CompilerParams(use_tc_tiling_on_sc=False),
        scratch_types=(
            [pltpu.VMEM((CH,), jnp.int32) for _ in range(NBUF)]
            + [pltpu.VMEM((CH, emb), jnp.float32) for _ in range(NBUF)]
            + [pltpu.VMEM((seq, emb), jnp.float32)]
            + [pltpu.SemaphoreType.DMA for _ in range(2 * NBUF)]
        ),
    )(x, embed_w, pos_w)
    return out
